# baseline (speedup 1.0000x reference)
import functools

import jax
import jax.numpy as jnp
from jax import lax
from jax.experimental import pallas as pl
from jax.experimental.pallas import tpu as pltpu
from jax.experimental.pallas import tpu_sc as plsc

_BATCH = 4096
_HIDDEN = 1024


@functools.cache
def _build(batch: int, hidden: int, n_rows: int, dtype):
    info = plsc.get_sparse_core_info()
    nc, ns = info.num_cores, info.num_subcores
    nw = nc * ns
    assert batch % nw == 0
    b_per_w = batch // nw
    chunk_sizes = (8, 24, 32, 32, 32)
    buf_of = (0, 1, 2, 3, 2)
    offs = (0, 8, 32, 64, 96)
    assert sum(chunk_sizes) == b_per_w
    mesh = plsc.VectorSubcoreMesh(core_axis_name="c", subcore_axis_name="s")

    @functools.partial(
        pl.kernel,
        mesh=mesh,
        out_type=jax.ShapeDtypeStruct((batch, hidden), dtype),
        scratch_types=[
            pltpu.VMEM((b_per_w,), jnp.int32),
            pltpu.VMEM((8, hidden), dtype),
            pltpu.VMEM((24, hidden), dtype),
            pltpu.VMEM((32, hidden), dtype),
            pltpu.VMEM((32, hidden), dtype),
            pltpu.SemaphoreType.DMA,
            pltpu.SemaphoreType.DMA,
            pltpu.SemaphoreType.DMA,
            pltpu.SemaphoreType.DMA,
            pltpu.SemaphoreType.DMA,
            pltpu.SemaphoreType.DMA,
            pltpu.SemaphoreType.DMA,
            pltpu.SemaphoreType.DMA,
        ],
    )
    def emb(
        table_hbm, idx_hbm, out_hbm,
        idx_v, buf0, buf1, buf2, buf3,
        gs0, gs1, gs2, gs3, ss0, ss1, ss2, ss3,
    ):
        wid = lax.axis_index("s") * nc + lax.axis_index("c")
        base = wid * b_per_w
        pltpu.sync_copy(idx_hbm.at[pl.ds(base, b_per_w)], idx_v)

        bufs = (buf0, buf1, buf2, buf3)
        gsems = (gs0, gs1, gs2, gs3)
        ssems = (ss0, ss1, ss2, ss3)

        def gather(c):
            b = buf_of[c]
            return pltpu.async_copy(
                table_hbm.at[idx_v.at[pl.ds(offs[c], chunk_sizes[c])]],
                bufs[b], gsems[b],
            )

        def store(c):
            b = buf_of[c]
            return pltpu.async_copy(
                bufs[b], out_hbm.at[pl.ds(base + offs[c], chunk_sizes[c])],
                ssems[b],
            )

        g0, g1, g2, g3 = gather(0), gather(1), gather(2), gather(3)
        g0.wait()
        s0 = store(0)
        g1.wait()
        s1 = store(1)
        g2.wait()
        s2 = store(2)
        s2.wait()
        g4 = gather(4)
        g3.wait()
        s3 = store(3)
        g4.wait()
        s4 = store(4)
        s0.wait()
        s1.wait()
        s3.wait()
        s4.wait()

    return emb


def kernel(labels, embedding_table):
    n_rows, hidden = embedding_table.shape
    emb = _build(labels.shape[0], hidden, n_rows, embedding_table.dtype)
    return emb(embedding_table, labels)


# speedup vs baseline: 1.0441x; 1.0441x over previous
import functools

import jax
import jax.numpy as jnp
from jax import lax
from jax.experimental import pallas as pl
from jax.experimental.pallas import tpu as pltpu
from jax.experimental.pallas import tpu_sc as plsc

_BATCH = 4096
_HIDDEN = 1024


@functools.cache
def _build(batch: int, hidden: int, n_rows: int, dtype):
    info = plsc.get_sparse_core_info()
    nc, ns = info.num_cores, info.num_subcores
    nw = nc * ns
    assert batch % nw == 0
    b_per_w = batch // nw
    chunk_sizes = (16, 48, 56, 8)
    buf_of = (0, 1, 2, 0)
    offs = (0, 16, 64, 120)
    assert sum(chunk_sizes) == b_per_w
    mesh = plsc.VectorSubcoreMesh(core_axis_name="c", subcore_axis_name="s")

    @functools.partial(
        pl.kernel,
        mesh=mesh,
        out_type=jax.ShapeDtypeStruct((batch, hidden), dtype),
        scratch_types=[
            pltpu.VMEM((b_per_w,), jnp.int32),
            pltpu.VMEM((16, hidden), dtype),
            pltpu.VMEM((48, hidden), dtype),
            pltpu.VMEM((56, hidden), dtype),
            pltpu.SemaphoreType.DMA,
            pltpu.SemaphoreType.DMA,
            pltpu.SemaphoreType.DMA,
            pltpu.SemaphoreType.DMA,
            pltpu.SemaphoreType.DMA,
            pltpu.SemaphoreType.DMA,
        ],
    )
    def emb(
        table_hbm, idx_hbm, out_hbm,
        idx_v, buf0, buf1, buf2,
        gs0, gs1, gs2, ss0, ss1, ss2,
    ):
        wid = lax.axis_index("s") * nc + lax.axis_index("c")
        base = wid * b_per_w
        pltpu.sync_copy(idx_hbm.at[pl.ds(base, b_per_w)], idx_v)

        bufs = (buf0, buf1, buf2)
        gsems = (gs0, gs1, gs2)
        ssems = (ss0, ss1, ss2)

        def gather(c):
            b = buf_of[c]
            dst = bufs[b]
            if chunk_sizes[c] != dst.shape[0]:
                dst = dst.at[pl.ds(0, chunk_sizes[c])]
            return pltpu.async_copy(
                table_hbm.at[idx_v.at[pl.ds(offs[c], chunk_sizes[c])]],
                dst, gsems[b],
            )

        def store(c):
            b = buf_of[c]
            src = bufs[b]
            if chunk_sizes[c] != src.shape[0]:
                src = src.at[pl.ds(0, chunk_sizes[c])]
            return pltpu.async_copy(
                src, out_hbm.at[pl.ds(base + offs[c], chunk_sizes[c])],
                ssems[b],
            )

        g0, g1, g2 = gather(0), gather(1), gather(2)
        g0.wait()
        s0 = store(0)
        g1.wait()
        s1 = store(1)
        g2.wait()
        s2 = store(2)
        s0.wait()
        g3 = gather(3)
        g3.wait()
        s3 = store(3)
        s1.wait()
        s2.wait()
        s3.wait()

    return emb


def kernel(labels, embedding_table):
    n_rows, hidden = embedding_table.shape
    emb = _build(labels.shape[0], hidden, n_rows, embedding_table.dtype)
    return emb(embedding_table, labels)
